# baseline (speedup 1.0000x reference)
import functools

import jax
import jax.numpy as jnp
from jax import lax
from jax.experimental import pallas as pl
from jax.experimental.pallas import tpu as pltpu
from jax.experimental.pallas import tpu_sc as plsc

_NUM_TYPES = 100
_NUM_REPS = 3
_EMBED = 64
_ROW = (_NUM_TYPES + _NUM_REPS) * _EMBED

_NC = 2
_NS = 16
_NW = _NC * _NS

_SROWS = 128
_RPS = _SROWS // _NS


def _bcast_sc(table, batch):
    b_per_w = batch // _NW
    n_chunks = b_per_w // _SROWS
    mesh = plsc.VectorSubcoreMesh(core_axis_name="c", subcore_axis_name="s")

    @functools.partial(
        pl.kernel,
        mesh=mesh,
        out_type=jax.ShapeDtypeStruct((batch, _ROW), jnp.float32),
        scratch_types=[
            pltpu.VMEM((1, _ROW), jnp.float32),
            pltpu.VMEM_SHARED((_SROWS, _ROW), jnp.float32),
            pltpu.SemaphoreType.DMA,
            pltpu.SemaphoreType.DMA,
        ],
    )
    def body(table_hbm, out_hbm, buf, shared, load_sem, store_sem):
        sid = lax.axis_index("s")
        wid = sid * _NC + lax.axis_index("c")
        base = wid * b_per_w
        pltpu.async_copy(table_hbm, buf.at[0], load_sem).wait()
        stage = [
            pltpu.async_copy(buf.at[0], shared.at[sid * _RPS + i], load_sem)
            for i in range(_RPS)
        ]
        for c in stage:
            c.wait()
        plsc.subcore_barrier()
        stores = [
            pltpu.async_copy(
                shared, out_hbm.at[pl.ds(base + j * _SROWS, _SROWS)], store_sem
            )
            for j in range(n_chunks)
        ]
        for c in stores:
            c.wait()

    return body(table)


def kernel(features, type_table, rep_table):
    batch = features.shape[0]
    table = jnp.concatenate(
        [type_table.reshape(-1), rep_table.reshape(-1)]
    )
    out = _bcast_sc(table, batch)
    return out.reshape(batch, _NUM_TYPES + _NUM_REPS, _EMBED)


# speedup vs baseline: 1.0845x; 1.0845x over previous
import functools

import jax
import jax.numpy as jnp
from jax import lax
from jax.experimental import pallas as pl
from jax.experimental.pallas import tpu as pltpu
from jax.experimental.pallas import tpu_sc as plsc

_NUM_TYPES = 100
_NUM_REPS = 3
_EMBED = 64
_ROW = (_NUM_TYPES + _NUM_REPS) * _EMBED

_NC = 2
_NS = 16
_NW = _NC * _NS

_CHUNK = 16


def _bcast_sc(table, batch):
    b_per_w = batch // _NW
    n_chunks = b_per_w // _CHUNK
    mesh = plsc.VectorSubcoreMesh(core_axis_name="c", subcore_axis_name="s")

    @functools.partial(
        pl.kernel,
        mesh=mesh,
        out_type=jax.ShapeDtypeStruct((batch, _ROW), jnp.float32),
        scratch_types=[
            pltpu.VMEM((_CHUNK, _ROW), jnp.float32),
            pltpu.SemaphoreType.DMA,
            pltpu.SemaphoreType.DMA,
        ],
    )
    def body(table_hbm, out_hbm, buf, load_sem, store_sem):
        wid = lax.axis_index("s") * _NC + lax.axis_index("c")
        base = wid * b_per_w
        loads = [
            pltpu.async_copy(table_hbm, buf.at[i], load_sem)
            for i in range(_CHUNK)
        ]
        for c in loads:
            c.wait()
        stores = [
            pltpu.async_copy(
                buf, out_hbm.at[pl.ds(base + j * _CHUNK, _CHUNK)], store_sem
            )
            for j in range(n_chunks)
        ]
        for c in stores:
            c.wait()

    return body(table)


def kernel(features, type_table, rep_table):
    batch = features.shape[0]
    table = jnp.concatenate(
        [type_table.reshape(-1), rep_table.reshape(-1)]
    )
    out = _bcast_sc(table, batch)
    return out.reshape(batch, _NUM_TYPES + _NUM_REPS, _EMBED)
